# baseline (speedup 1.0000x reference)
import jax
import jax.numpy as jnp
from jax import lax
from jax.experimental import pallas as pl
from jax.experimental.pallas import tpu as pltpu
from jax.experimental.pallas import tpu_sc as plsc

_NS, _L = 16, 16
_NCHUNK = 4


def _make_body(B, N, D, bpw):
    cs = bpw // _NCHUNK

    def body(x_hbm, idx_hbm, out_hbm, idx_v, rows_v, *sems):
        isem = sems[0]
        gsem = sems[1:1 + _NCHUNK]
        ssem = sems[1 + _NCHUNK:]
        base = lax.axis_index("s") * bpw
        iload = pltpu.async_copy(idx_hbm.at[pl.ds(base, bpw)], idx_v, isem)
        iload.wait()
        gathers = []
        for c in range(_NCHUNK):
            rows = pl.ds(c * cs, cs)
            for j in range(cs // _L):
                seg = pl.ds(c * cs + j * _L, _L)
                batch_ids = (base + c * cs + j * _L) + lax.broadcasted_iota(
                    jnp.int32, (_L,), 0
                )
                idx_v[seg] = batch_ids * N + idx_v[seg]
            gathers.append(
                pltpu.async_copy(
                    x_hbm.at[idx_v.at[rows]], rows_v.at[rows], gsem[c]
                )
            )
        stores = []
        for c in range(_NCHUNK):
            rows = pl.ds(c * cs, cs)
            gathers[c].wait()
            stores.append(
                pltpu.async_copy(
                    rows_v.at[rows], out_hbm.at[pl.ds(base + c * cs, cs)],
                    ssem[c],
                )
            )
        for s in stores:
            s.wait()

    return body


def kernel(x, edge_index, agent_nodes):
    del edge_index
    B, N, D = x.shape
    bpw = B // _NS
    x_flat = x.reshape(B * N, D)
    idx32 = agent_nodes.astype(jnp.int32)
    mesh = plsc.VectorSubcoreMesh(
        core_axis_name="c", subcore_axis_name="s", num_cores=1
    )
    k = pl.kernel(
        _make_body(B, N, D, bpw),
        mesh=mesh,
        out_type=jax.ShapeDtypeStruct((B, D), jnp.float32),
        scratch_types=[
            pltpu.VMEM((bpw,), jnp.int32),
            pltpu.VMEM((bpw, D), jnp.float32),
        ] + [pltpu.SemaphoreType.DMA] * (1 + 2 * _NCHUNK),
    )
    return k(x_flat, idx32)


# speedup vs baseline: 1.0116x; 1.0116x over previous
import jax
import jax.numpy as jnp
from jax import lax
from jax.experimental import pallas as pl
from jax.experimental.pallas import tpu as pltpu
from jax.experimental.pallas import tpu_sc as plsc

_NS, _L = 16, 16


def _make_body(B, N, D, bpw):
    def body(x_hbm, idx_hbm, out_hbm, idx_v, rows_v, sem):
        base = lax.axis_index("s") * bpw
        pltpu.sync_copy(idx_hbm.at[pl.ds(base, bpw)], idx_v)
        for j in range(bpw // _L):
            seg = pl.ds(j * _L, _L)
            batch_ids = (base + j * _L) + lax.broadcasted_iota(
                jnp.int32, (_L,), 0
            )
            idx_v[seg] = batch_ids * N + idx_v[seg]
        pltpu.async_copy(x_hbm.at[idx_v], rows_v, sem).wait()
        pltpu.sync_copy(rows_v, out_hbm.at[pl.ds(base, bpw)])

    return body


def kernel(x, edge_index, agent_nodes):
    del edge_index
    B, N, D = x.shape
    bpw = B // _NS
    x_flat = x.reshape(B * N, D)
    idx32 = agent_nodes.astype(jnp.int32)
    mesh = plsc.VectorSubcoreMesh(
        core_axis_name="c", subcore_axis_name="s", num_cores=1
    )
    k = pl.kernel(
        _make_body(B, N, D, bpw),
        mesh=mesh,
        out_type=jax.ShapeDtypeStruct((B, D), jnp.float32),
        scratch_types=[
            pltpu.VMEM((bpw,), jnp.int32),
            pltpu.VMEM((bpw, D), jnp.float32),
            pltpu.SemaphoreType.DMA,
        ],
    )
    return k(x_flat, idx32)
